# baseline (speedup 1.0000x reference)
import functools

import jax
import jax.numpy as jnp
from jax import lax
from jax.experimental import pallas as pl
from jax.experimental.pallas import tpu as pltpu
from jax.experimental.pallas import tpu_sc as plsc

NUM_EXPERTS = 8
TOPK = 2
HIDDEN = 2048
INTER = 4096
NTOKENS = 4096
NROWS = NTOKENS * TOPK

BM = 256
BN = 1024
P = NROWS + NUM_EXPERTS * BM
NB = P // BM
NN = HIDDEN // BN

NC = 2
NS = 16
NW = NC * NS

GCH = 8
RPW = P // NW
NPAIR = RPW // (2 * GCH)
TPW = NTOKENS // NW
TCH = 8

_mesh = plsc.VectorSubcoreMesh(core_axis_name="c", subcore_axis_name="s")
_sc_params = pltpu.CompilerParams(use_tc_tiling_on_sc=True)


@functools.partial(
    pl.kernel,
    mesh=_mesh,
    out_type=jax.ShapeDtypeStruct((P, INTER), jnp.float32),
    scratch_types=[
        pltpu.VMEM((RPW,), jnp.int32),
        pltpu.VMEM((GCH, INTER), jnp.float32),
        pltpu.VMEM((GCH, INTER), jnp.float32),
        pltpu.SemaphoreType.DMA,
        pltpu.SemaphoreType.DMA,
        pltpu.SemaphoreType.DMA,
        pltpu.SemaphoreType.DMA,
    ],
    compiler_params=_sc_params,
)
def _gather_rows(x_hbm, gidx_hbm, out_hbm, idx_v, buf0, buf1, g0, g1, w0, w1):
    wid = lax.axis_index("s") * NC + lax.axis_index("c")
    base = wid * RPW
    pltpu.sync_copy(gidx_hbm.at[pl.ds(base, RPW)], idx_v)

    def g_start(c, buf, sem):
        pltpu.async_copy(x_hbm.at[idx_v.at[pl.ds(c * GCH, GCH)]], buf, sem)

    def g_wait(c, buf, sem):
        pltpu.make_async_copy(x_hbm.at[idx_v.at[pl.ds(c * GCH, GCH)]], buf, sem).wait()

    def w_start(c, buf, sem):
        pltpu.async_copy(buf, out_hbm.at[pl.ds(base + c * GCH, GCH)], sem)

    def w_wait(c, buf, sem):
        pltpu.make_async_copy(buf, out_hbm.at[pl.ds(base + c * GCH, GCH)], sem).wait()

    g_start(0, buf0, g0)
    g_start(1, buf1, g1)

    def pair(g, _):
        c0 = 2 * g
        c1 = c0 + 1
        g_wait(c0, buf0, g0)
        w_start(c0, buf0, w0)
        g_wait(c1, buf1, g1)
        w_start(c1, buf1, w1)

        @pl.when(g < NPAIR - 1)
        def _():
            w_wait(c0, buf0, w0)
            g_start(c0 + 2, buf0, g0)
            w_wait(c1, buf1, w1)
            g_start(c1 + 2, buf1, g1)

        return ()

    lax.fori_loop(0, NPAIR, pair, ())
    w_wait(2 * NPAIR - 2, buf0, w0)
    w_wait(2 * NPAIR - 1, buf1, w1)


def _mm_body(be_ref, x_ref, w_ref, sw_ref, o_ref):
    acc = jnp.dot(x_ref[...], w_ref[0], preferred_element_type=jnp.float32)
    o_ref[...] = acc * sw_ref[0, 0, :][:, None]


def _grouped_matmul(block_expert, xs, w, sw):
    grid_spec = pltpu.PrefetchScalarGridSpec(
        num_scalar_prefetch=1,
        grid=(NN, NB),
        in_specs=[
            pl.BlockSpec((BM, INTER), lambda n, m, be: (m, 0)),
            pl.BlockSpec((1, INTER, BN), lambda n, m, be: (be[m], 0, n)),
            pl.BlockSpec((1, 1, BM), lambda n, m, be: (m, 0, 0)),
        ],
        out_specs=pl.BlockSpec((BM, BN), lambda n, m, be: (m, n)),
    )
    return pl.pallas_call(
        _mm_body,
        grid_spec=grid_spec,
        out_shape=jax.ShapeDtypeStruct((P, HIDDEN), jnp.float32),
        compiler_params=pltpu.CompilerParams(
            dimension_semantics=("arbitrary", "arbitrary"),
        ),
    )(block_expert, xs, w, sw)


@functools.partial(
    pl.kernel,
    mesh=_mesh,
    out_type=jax.ShapeDtypeStruct((NTOKENS, HIDDEN), jnp.float32),
    scratch_types=[
        pltpu.VMEM((2 * TPW,), jnp.int32),
        pltpu.VMEM((2 * TCH, HIDDEN), jnp.float32),
        pltpu.VMEM((TCH, HIDDEN), jnp.float32),
        pltpu.SemaphoreType.DMA,
    ],
    compiler_params=_sc_params,
)
def _combine(y_hbm, pidx_hbm, out_hbm, idx_v, rows_v, out_v, sem):
    wid = lax.axis_index("s") * NC + lax.axis_index("c")
    tbase = wid * TPW
    pltpu.sync_copy(pidx_hbm.at[pl.ds(tbase * 2, 2 * TPW)], idx_v)

    def chunk(c, _):
        t0 = tbase + c * TCH
        pltpu.async_copy(
            y_hbm.at[idx_v.at[pl.ds(c * 2 * TCH, 2 * TCH)]], rows_v, sem)
        pltpu.make_async_copy(
            y_hbm.at[idx_v.at[pl.ds(c * 2 * TCH, 2 * TCH)]], rows_v, sem).wait()

        def jloop(j, _):
            off = j * 16
            for i in range(TCH):
                out_v[i, pl.ds(off, 16)] = (
                    rows_v[i, pl.ds(off, 16)] + rows_v[i + TCH, pl.ds(off, 16)]
                )
            return ()

        lax.fori_loop(0, HIDDEN // 16, jloop, ())
        pltpu.sync_copy(out_v, out_hbm.at[pl.ds(t0, TCH)])
        return ()

    lax.fori_loop(0, TPW // TCH, chunk, ())


def kernel(intermediate_states, w, topk_ids, topk_weight):
    flat_ids = topk_ids.reshape(-1)
    flat_w = topk_weight.reshape(-1)

    onehot = (flat_ids[:, None] == jnp.arange(NUM_EXPERTS, dtype=jnp.int32)[None, :])
    counts = jnp.sum(onehot.astype(jnp.int32), axis=0)
    padded = ((counts + BM - 1) // BM) * BM
    ends = jnp.cumsum(padded)
    starts = ends - padded
    cstart = jnp.concatenate(
        [jnp.zeros((1,), jnp.int32), jnp.cumsum(counts)[:-1].astype(jnp.int32)])
    shift = starts - cstart

    block_expert = jnp.clip(
        jnp.searchsorted(ends, jnp.arange(NB, dtype=jnp.int32) * BM, side="right"),
        0, NUM_EXPERTS - 1).astype(jnp.int32)

    order = jnp.argsort(flat_ids, stable=True).astype(jnp.int32)
    slot_e = jnp.repeat(block_expert, BM)
    s = jnp.arange(P, dtype=jnp.int32)
    srank = s - shift[slot_e]
    valid = s < starts[slot_e] + counts[slot_e]
    gidx = jnp.where(valid, order[jnp.clip(srank, 0, NROWS - 1)], 0)
    sw = jnp.where(valid, flat_w[gidx], 0.0)

    inv_order = jnp.argsort(order, stable=True).astype(jnp.int32)
    pos = inv_order + shift[flat_ids]

    pidx = pos.reshape(NTOKENS // TCH, TCH, TOPK).transpose(0, 2, 1).reshape(-1)
    pidx = pidx.astype(jnp.int32)

    xs = _gather_rows(intermediate_states, gidx)

    y = _grouped_matmul(block_expert, xs, w, sw.reshape(NB, 1, BM))

    return _combine(y, pidx)


# speedup vs baseline: 1.2373x; 1.2373x over previous
import functools

import jax
import jax.numpy as jnp
from jax import lax
from jax.experimental import pallas as pl
from jax.experimental.pallas import tpu as pltpu
from jax.experimental.pallas import tpu_sc as plsc

NUM_EXPERTS = 8
TOPK = 2
HIDDEN = 2048
INTER = 4096
NTOKENS = 4096
NROWS = NTOKENS * TOPK

BM = 256
P = NROWS + NUM_EXPERTS * BM
NB = P // BM

NC = 2
NS = 16
NW = NC * NS

TPW = NTOKENS // NW
TCH = 8

_mesh = plsc.VectorSubcoreMesh(core_axis_name="c", subcore_axis_name="s")
_sc_params = pltpu.CompilerParams(use_tc_tiling_on_sc=True)


def _mm_body(gidx_ref, be_ref, x_hbm, w_hbm, sw_ref, o_ref, xbuf, wbuf, sems, wsem):
    m = pl.program_id(0)

    new_w = jnp.where(m == 0, True,
                      be_ref[m] != be_ref[jnp.maximum(m - 1, 0)])

    @pl.when(new_w)
    def _():
        pltpu.make_async_copy(w_hbm.at[be_ref[m]], wbuf, wsem).start()

    def transfer(slot, blk, start):
        def one(i, _):
            r = gidx_ref[blk * BM + i]

            @pl.when(r >= 0)
            def _():
                cp = pltpu.make_async_copy(
                    x_hbm.at[pl.ds(r, 1)],
                    xbuf.at[slot, pl.ds(i, 1)],
                    sems.at[slot])
                if start:
                    cp.start()
                else:
                    cp.wait()
            return ()

        lax.fori_loop(0, BM, one, ())

    @pl.when(m == 0)
    def _():
        transfer(0, 0, True)

    @pl.when(m + 1 < NB)
    def _():
        transfer((m + 1) % 2, m + 1, True)

    transfer(m % 2, m, False)

    @pl.when(new_w)
    def _():
        pltpu.make_async_copy(w_hbm.at[be_ref[m]], wbuf, wsem).wait()

    def do_dot(slot):
        acc = jnp.dot(xbuf[slot], wbuf[...], preferred_element_type=jnp.float32)
        o_ref[...] = acc * sw_ref[0, 0, :][:, None]

    @pl.when(m % 2 == 0)
    def _():
        do_dot(0)

    @pl.when(m % 2 == 1)
    def _():
        do_dot(1)


def _grouped_matmul(gidx, block_expert, x, w, sw):
    grid_spec = pltpu.PrefetchScalarGridSpec(
        num_scalar_prefetch=2,
        grid=(NB,),
        in_specs=[
            pl.BlockSpec(memory_space=pl.ANY),
            pl.BlockSpec(memory_space=pl.ANY),
            pl.BlockSpec((1, 1, BM), lambda m, g, be: (m, 0, 0)),
        ],
        out_specs=pl.BlockSpec((BM, HIDDEN), lambda m, g, be: (m, 0)),
        scratch_shapes=[
            pltpu.VMEM((2, BM, INTER), jnp.float32),
            pltpu.VMEM((INTER, HIDDEN), jnp.float32),
            pltpu.SemaphoreType.DMA((2,)),
            pltpu.SemaphoreType.DMA,
        ],
    )
    return pl.pallas_call(
        _mm_body,
        grid_spec=grid_spec,
        out_shape=jax.ShapeDtypeStruct((P, HIDDEN), jnp.float32),
        compiler_params=pltpu.CompilerParams(
            dimension_semantics=("arbitrary",),
            vmem_limit_bytes=100 * 1024 * 1024,
        ),
    )(gidx, block_expert, x, w, sw)


@functools.partial(
    pl.kernel,
    mesh=_mesh,
    out_type=jax.ShapeDtypeStruct((NTOKENS, HIDDEN), jnp.float32),
    scratch_types=[
        pltpu.VMEM((2 * TPW,), jnp.int32),
        pltpu.VMEM((2 * TCH, HIDDEN), jnp.float32),
        pltpu.VMEM((TCH, HIDDEN), jnp.float32),
        pltpu.SemaphoreType.DMA,
    ],
    compiler_params=_sc_params,
)
def _combine(y_hbm, pidx_hbm, out_hbm, idx_v, rows_v, out_v, sem):
    wid = lax.axis_index("s") * NC + lax.axis_index("c")
    tbase = wid * TPW
    pltpu.sync_copy(pidx_hbm.at[pl.ds(tbase * 2, 2 * TPW)], idx_v)

    def chunk(c, _):
        t0 = tbase + c * TCH
        pltpu.async_copy(
            y_hbm.at[idx_v.at[pl.ds(c * 2 * TCH, 2 * TCH)]], rows_v, sem)
        pltpu.make_async_copy(
            y_hbm.at[idx_v.at[pl.ds(c * 2 * TCH, 2 * TCH)]], rows_v, sem).wait()

        def jloop(j, _):
            off = j * 16
            for i in range(TCH):
                out_v[i, pl.ds(off, 16)] = (
                    rows_v[i, pl.ds(off, 16)] + rows_v[i + TCH, pl.ds(off, 16)]
                )
            return ()

        lax.fori_loop(0, HIDDEN // 16, jloop, ())
        pltpu.sync_copy(out_v, out_hbm.at[pl.ds(t0, TCH)])
        return ()

    lax.fori_loop(0, TPW // TCH, chunk, ())


def kernel(intermediate_states, w, topk_ids, topk_weight):
    flat_ids = topk_ids.reshape(-1)
    flat_w = topk_weight.reshape(-1)

    onehot = (flat_ids[:, None] == jnp.arange(NUM_EXPERTS, dtype=jnp.int32)[None, :])
    counts = jnp.sum(onehot.astype(jnp.int32), axis=0)
    padded = ((counts + BM - 1) // BM) * BM
    ends = jnp.cumsum(padded)
    starts = ends - padded
    cstart = jnp.concatenate(
        [jnp.zeros((1,), jnp.int32), jnp.cumsum(counts)[:-1].astype(jnp.int32)])
    shift = starts - cstart

    block_expert = jnp.clip(
        jnp.searchsorted(ends, jnp.arange(NB, dtype=jnp.int32) * BM, side="right"),
        0, NUM_EXPERTS - 1).astype(jnp.int32)

    order = jnp.argsort(flat_ids, stable=True).astype(jnp.int32)
    slot_e = jnp.repeat(block_expert, BM)
    s = jnp.arange(P, dtype=jnp.int32)
    srank = s - shift[slot_e]
    valid = s < starts[slot_e] + counts[slot_e]
    gidx = jnp.where(valid, order[jnp.clip(srank, 0, NROWS - 1)], -1)
    sw = jnp.where(valid, flat_w[jnp.maximum(gidx, 0)], 0.0)

    inv_order = jnp.argsort(order, stable=True).astype(jnp.int32)
    pos = inv_order + shift[flat_ids]

    pidx = pos.reshape(NTOKENS // TCH, TCH, TOPK).transpose(0, 2, 1).reshape(-1)
    pidx = pidx.astype(jnp.int32)

    y = _grouped_matmul(gidx, block_expert, intermediate_states, w,
                        sw.reshape(NB, 1, BM))

    return _combine(y, pidx)


# speedup vs baseline: 1.2485x; 1.0090x over previous
import functools

import jax
import jax.numpy as jnp
from jax import lax
from jax.experimental import pallas as pl
from jax.experimental.pallas import tpu as pltpu
from jax.experimental.pallas import tpu_sc as plsc

NUM_EXPERTS = 8
TOPK = 2
HIDDEN = 2048
INTER = 4096
NTOKENS = 4096
NROWS = NTOKENS * TOPK

BM = 128
P = NROWS + NUM_EXPERTS * BM
NB = P // BM

NC = 2
NS = 16
NW = NC * NS

TPW = NTOKENS // NW
TCH = 8

_mesh = plsc.VectorSubcoreMesh(core_axis_name="c", subcore_axis_name="s")
_sc_params = pltpu.CompilerParams(use_tc_tiling_on_sc=True)


def _mm_body(gidx_ref, be_ref, x_hbm, w_hbm, sw_ref, o_ref, xbuf, wbuf, wbf, sems, wsem):
    m = pl.program_id(0)

    new_w = jnp.where(m == 0, True,
                      be_ref[m] != be_ref[jnp.maximum(m - 1, 0)])

    @pl.when(new_w)
    def _():
        pltpu.make_async_copy(w_hbm.at[be_ref[m]], wbuf, wsem).start()

    def transfer(slot, blk, start):
        def one(i, _):
            r = gidx_ref[blk * BM + i]

            @pl.when(r >= 0)
            def _():
                cp = pltpu.make_async_copy(
                    x_hbm.at[pl.ds(r, 1)],
                    xbuf.at[slot, pl.ds(i, 1)],
                    sems.at[slot])
                if start:
                    cp.start()
                else:
                    cp.wait()
            return ()

        lax.fori_loop(0, BM, one, ())

    @pl.when(m == 0)
    def _():
        transfer(0, 0, True)

    @pl.when(m + 1 < NB)
    def _():
        transfer((m + 1) % 2, m + 1, True)

    transfer(m % 2, m, False)

    @pl.when(new_w)
    def _():
        pltpu.make_async_copy(w_hbm.at[be_ref[m]], wbuf, wsem).wait()
        wbf[...] = wbuf[...].astype(jnp.bfloat16)

    def do_dot(slot):
        acc = jnp.dot(xbuf[slot].astype(jnp.bfloat16), wbf[...],
                      preferred_element_type=jnp.float32)
        o_ref[...] = acc * sw_ref[0, 0, :][:, None]

    @pl.when(m % 2 == 0)
    def _():
        do_dot(0)

    @pl.when(m % 2 == 1)
    def _():
        do_dot(1)


def _grouped_matmul(gidx, block_expert, x, w, sw):
    grid_spec = pltpu.PrefetchScalarGridSpec(
        num_scalar_prefetch=2,
        grid=(NB,),
        in_specs=[
            pl.BlockSpec(memory_space=pl.ANY),
            pl.BlockSpec(memory_space=pl.ANY),
            pl.BlockSpec((1, 1, BM), lambda m, g, be: (m, 0, 0)),
        ],
        out_specs=pl.BlockSpec((BM, HIDDEN), lambda m, g, be: (m, 0)),
        scratch_shapes=[
            pltpu.VMEM((2, BM, INTER), jnp.float32),
            pltpu.VMEM((INTER, HIDDEN), jnp.float32),
            pltpu.VMEM((INTER, HIDDEN), jnp.bfloat16),
            pltpu.SemaphoreType.DMA((2,)),
            pltpu.SemaphoreType.DMA,
        ],
    )
    return pl.pallas_call(
        _mm_body,
        grid_spec=grid_spec,
        out_shape=jax.ShapeDtypeStruct((P, HIDDEN), jnp.float32),
        compiler_params=pltpu.CompilerParams(
            dimension_semantics=("arbitrary",),
            vmem_limit_bytes=100 * 1024 * 1024,
        ),
    )(gidx, block_expert, x, w, sw)


@functools.partial(
    pl.kernel,
    mesh=_mesh,
    out_type=jax.ShapeDtypeStruct((NTOKENS, HIDDEN), jnp.float32),
    scratch_types=[
        pltpu.VMEM((2 * TPW,), jnp.int32),
        pltpu.VMEM((2 * TCH, HIDDEN), jnp.float32),
        pltpu.VMEM((TCH, HIDDEN), jnp.float32),
        pltpu.SemaphoreType.DMA,
    ],
    compiler_params=_sc_params,
)
def _combine(y_hbm, pidx_hbm, out_hbm, idx_v, rows_v, out_v, sem):
    wid = lax.axis_index("s") * NC + lax.axis_index("c")
    tbase = wid * TPW
    pltpu.sync_copy(pidx_hbm.at[pl.ds(tbase * 2, 2 * TPW)], idx_v)

    def chunk(c, _):
        t0 = tbase + c * TCH
        pltpu.async_copy(
            y_hbm.at[idx_v.at[pl.ds(c * 2 * TCH, 2 * TCH)]], rows_v, sem)
        pltpu.make_async_copy(
            y_hbm.at[idx_v.at[pl.ds(c * 2 * TCH, 2 * TCH)]], rows_v, sem).wait()

        def jloop(j, _):
            off = j * 16
            for i in range(TCH):
                out_v[i, pl.ds(off, 16)] = (
                    rows_v[i, pl.ds(off, 16)] + rows_v[i + TCH, pl.ds(off, 16)]
                )
            return ()

        lax.fori_loop(0, HIDDEN // 16, jloop, ())
        pltpu.sync_copy(out_v, out_hbm.at[pl.ds(t0, TCH)])
        return ()

    lax.fori_loop(0, TPW // TCH, chunk, ())


def kernel(intermediate_states, w, topk_ids, topk_weight):
    flat_ids = topk_ids.reshape(-1)
    flat_w = topk_weight.reshape(-1)

    onehot = (flat_ids[:, None] == jnp.arange(NUM_EXPERTS, dtype=jnp.int32)[None, :])
    counts = jnp.sum(onehot.astype(jnp.int32), axis=0)
    padded = ((counts + BM - 1) // BM) * BM
    ends = jnp.cumsum(padded)
    starts = ends - padded
    cstart = jnp.concatenate(
        [jnp.zeros((1,), jnp.int32), jnp.cumsum(counts)[:-1].astype(jnp.int32)])
    shift = starts - cstart

    block_expert = jnp.clip(
        jnp.searchsorted(ends, jnp.arange(NB, dtype=jnp.int32) * BM, side="right"),
        0, NUM_EXPERTS - 1).astype(jnp.int32)

    order = jnp.argsort(flat_ids, stable=True).astype(jnp.int32)
    slot_e = jnp.repeat(block_expert, BM)
    s = jnp.arange(P, dtype=jnp.int32)
    srank = s - shift[slot_e]
    valid = s < starts[slot_e] + counts[slot_e]
    gidx = jnp.where(valid, order[jnp.clip(srank, 0, NROWS - 1)], -1)
    sw = jnp.where(valid, flat_w[jnp.maximum(gidx, 0)], 0.0)

    inv_order = jnp.argsort(order, stable=True).astype(jnp.int32)
    pos = inv_order + shift[flat_ids]

    pidx = pos.reshape(NTOKENS // TCH, TCH, TOPK).transpose(0, 2, 1).reshape(-1)
    pidx = pidx.astype(jnp.int32)

    y = _grouped_matmul(gidx, block_expert, intermediate_states, w,
                        sw.reshape(NB, 1, BM))

    return _combine(y, pidx)


# speedup vs baseline: 1.3472x; 1.0790x over previous
import functools

import jax
import jax.numpy as jnp
from jax import lax
from jax.experimental import pallas as pl
from jax.experimental.pallas import tpu as pltpu
from jax.experimental.pallas import tpu_sc as plsc

NUM_EXPERTS = 8
TOPK = 2
HIDDEN = 2048
INTER = 4096
NTOKENS = 4096
NROWS = NTOKENS * TOPK

BM = 128
P = NROWS + NUM_EXPERTS * BM
NB = P // BM

NC = 2
NS = 16
NW = NC * NS

TPW = NTOKENS // NW
TCH = 8

_mesh = plsc.VectorSubcoreMesh(core_axis_name="c", subcore_axis_name="s")
_sc_params = pltpu.CompilerParams(use_tc_tiling_on_sc=True)


def _mm_body(gidx_ref, be_ref, x_hbm, w_hbm, sw_ref, o_ref, xbuf, wbuf, wbf, sems, wsem):
    m = pl.program_id(0)

    new_w = jnp.where(m == 0, True,
                      be_ref[m] != be_ref[jnp.maximum(m - 1, 0)])

    @pl.when(new_w)
    def _():
        pltpu.make_async_copy(w_hbm.at[be_ref[m]], wbuf, wsem).start()

    def transfer(slot, blk, start):
        if start:
            def one(i, _):
                r = jnp.maximum(gidx_ref[blk * BM + i], 0)
                pltpu.make_async_copy(
                    x_hbm.at[pl.ds(r, 1)],
                    xbuf.at[slot, pl.ds(i, 1)],
                    sems.at[slot]).start()
                return ()

            lax.fori_loop(0, BM, one, ())
        else:
            pltpu.make_async_copy(
                x_hbm.at[pl.ds(0, BM)], xbuf.at[slot], sems.at[slot]).wait()

    @pl.when(m == 0)
    def _():
        transfer(0, 0, True)

    @pl.when(m + 1 < NB)
    def _():
        transfer((m + 1) % 2, m + 1, True)

    transfer(m % 2, m, False)

    @pl.when(new_w)
    def _():
        pltpu.make_async_copy(w_hbm.at[be_ref[m]], wbuf, wsem).wait()
        wbf[...] = wbuf[...].astype(jnp.bfloat16)

    def do_dot(slot):
        acc = jnp.dot(xbuf[slot].astype(jnp.bfloat16), wbf[...],
                      preferred_element_type=jnp.float32)
        o_ref[...] = acc * sw_ref[0, 0, :][:, None]

    @pl.when(m % 2 == 0)
    def _():
        do_dot(0)

    @pl.when(m % 2 == 1)
    def _():
        do_dot(1)


def _grouped_matmul(gidx, block_expert, x, w, sw):
    grid_spec = pltpu.PrefetchScalarGridSpec(
        num_scalar_prefetch=2,
        grid=(NB,),
        in_specs=[
            pl.BlockSpec(memory_space=pl.ANY),
            pl.BlockSpec(memory_space=pl.ANY),
            pl.BlockSpec((1, 1, BM), lambda m, g, be: (m, 0, 0)),
        ],
        out_specs=pl.BlockSpec((BM, HIDDEN), lambda m, g, be: (m, 0)),
        scratch_shapes=[
            pltpu.VMEM((2, BM, INTER), jnp.float32),
            pltpu.VMEM((INTER, HIDDEN), jnp.float32),
            pltpu.VMEM((INTER, HIDDEN), jnp.bfloat16),
            pltpu.SemaphoreType.DMA((2,)),
            pltpu.SemaphoreType.DMA,
        ],
    )
    return pl.pallas_call(
        _mm_body,
        grid_spec=grid_spec,
        out_shape=jax.ShapeDtypeStruct((P, HIDDEN), jnp.float32),
        compiler_params=pltpu.CompilerParams(
            dimension_semantics=("arbitrary",),
            vmem_limit_bytes=100 * 1024 * 1024,
        ),
    )(gidx, block_expert, x, w, sw)


@functools.partial(
    pl.kernel,
    mesh=_mesh,
    out_type=jax.ShapeDtypeStruct((NTOKENS, HIDDEN), jnp.float32),
    scratch_types=[
        pltpu.VMEM((2 * TPW,), jnp.int32),
        pltpu.VMEM((2 * TCH, HIDDEN), jnp.float32),
        pltpu.VMEM((TCH, HIDDEN), jnp.float32),
        pltpu.SemaphoreType.DMA,
    ],
    compiler_params=_sc_params,
)
def _combine(y_hbm, pidx_hbm, out_hbm, idx_v, rows_v, out_v, sem):
    wid = lax.axis_index("s") * NC + lax.axis_index("c")
    tbase = wid * TPW
    pltpu.sync_copy(pidx_hbm.at[pl.ds(tbase * 2, 2 * TPW)], idx_v)

    def chunk(c, _):
        t0 = tbase + c * TCH
        pltpu.async_copy(
            y_hbm.at[idx_v.at[pl.ds(c * 2 * TCH, 2 * TCH)]], rows_v, sem)
        pltpu.make_async_copy(
            y_hbm.at[idx_v.at[pl.ds(c * 2 * TCH, 2 * TCH)]], rows_v, sem).wait()

        def jloop(j, _):
            off = j * 16
            for i in range(TCH):
                out_v[i, pl.ds(off, 16)] = (
                    rows_v[i, pl.ds(off, 16)] + rows_v[i + TCH, pl.ds(off, 16)]
                )
            return ()

        lax.fori_loop(0, HIDDEN // 16, jloop, ())
        pltpu.sync_copy(out_v, out_hbm.at[pl.ds(t0, TCH)])
        return ()

    lax.fori_loop(0, TPW // TCH, chunk, ())


def kernel(intermediate_states, w, topk_ids, topk_weight):
    flat_ids = topk_ids.reshape(-1)
    flat_w = topk_weight.reshape(-1)

    onehot = (flat_ids[:, None] == jnp.arange(NUM_EXPERTS, dtype=jnp.int32)[None, :])
    counts = jnp.sum(onehot.astype(jnp.int32), axis=0)
    padded = ((counts + BM - 1) // BM) * BM
    ends = jnp.cumsum(padded)
    starts = ends - padded
    cstart = jnp.concatenate(
        [jnp.zeros((1,), jnp.int32), jnp.cumsum(counts)[:-1].astype(jnp.int32)])
    shift = starts - cstart

    block_expert = jnp.clip(
        jnp.searchsorted(ends, jnp.arange(NB, dtype=jnp.int32) * BM, side="right"),
        0, NUM_EXPERTS - 1).astype(jnp.int32)

    order = jnp.argsort(flat_ids, stable=True).astype(jnp.int32)
    slot_e = jnp.repeat(block_expert, BM)
    s = jnp.arange(P, dtype=jnp.int32)
    srank = s - shift[slot_e]
    valid = s < starts[slot_e] + counts[slot_e]
    gidx = jnp.where(valid, order[jnp.clip(srank, 0, NROWS - 1)], -1)
    sw = jnp.where(valid, flat_w[jnp.maximum(gidx, 0)], 0.0)

    inv_order = jnp.argsort(order, stable=True).astype(jnp.int32)
    pos = inv_order + shift[flat_ids]

    pidx = pos.reshape(NTOKENS // TCH, TCH, TOPK).transpose(0, 2, 1).reshape(-1)
    pidx = pidx.astype(jnp.int32)

    y = _grouped_matmul(gidx, block_expert, intermediate_states, w,
                        sw.reshape(NB, 1, BM))

    return _combine(y, pidx)


# speedup vs baseline: 1.3858x; 1.0287x over previous
import functools

import jax
import jax.numpy as jnp
from jax import lax
from jax.experimental import pallas as pl
from jax.experimental.pallas import tpu as pltpu
from jax.experimental.pallas import tpu_sc as plsc

NUM_EXPERTS = 8
TOPK = 2
HIDDEN = 2048
INTER = 4096
NTOKENS = 4096
NROWS = NTOKENS * TOPK

BM = 128
P = NROWS + NUM_EXPERTS * BM
NB = P // BM

NC = 2
NS = 16
NW = NC * NS

TPW = NTOKENS // NW
TCH = 8

_mesh = plsc.VectorSubcoreMesh(core_axis_name="c", subcore_axis_name="s")
_sc_params = pltpu.CompilerParams(use_tc_tiling_on_sc=True)


def _mm_body(gidx_ref, be_ref, x_hbm, w_hbm, sw_ref, o_ref, xbuf, wbuf, wbf, sems, wsem):
    m = pl.program_id(0)

    new_w = jnp.where(m == 0, True,
                      be_ref[m] != be_ref[jnp.maximum(m - 1, 0)])

    @pl.when(new_w)
    def _():
        pltpu.make_async_copy(w_hbm.at[be_ref[m]], wbuf, wsem).start()

    def transfer(slot, blk, start):
        if start:
            def one(i8, _):
                for u in range(8):
                    i = i8 * 8 + u
                    r = jnp.maximum(gidx_ref[blk * BM + i], 0)
                    pltpu.make_async_copy(
                        x_hbm.at[pl.ds(r, 1)],
                        xbuf.at[slot, pl.ds(i, 1)],
                        sems.at[slot]).start()
                return ()

            lax.fori_loop(0, BM // 8, one, ())
        else:
            pltpu.make_async_copy(
                x_hbm.at[pl.ds(0, BM)], xbuf.at[slot], sems.at[slot]).wait()

    @pl.when(m == 0)
    def _():
        transfer(0, 0, True)

    @pl.when(m + 1 < NB)
    def _():
        transfer((m + 1) % 2, m + 1, True)

    transfer(m % 2, m, False)

    @pl.when(new_w)
    def _():
        pltpu.make_async_copy(w_hbm.at[be_ref[m]], wbuf, wsem).wait()
        wbf[...] = wbuf[...].astype(jnp.bfloat16)

    def do_dot(slot):
        acc = jnp.dot(xbuf[slot].astype(jnp.bfloat16), wbf[...],
                      preferred_element_type=jnp.float32)
        o_ref[...] = acc * sw_ref[0, 0, :][:, None]

    @pl.when(m % 2 == 0)
    def _():
        do_dot(0)

    @pl.when(m % 2 == 1)
    def _():
        do_dot(1)


def _grouped_matmul(gidx, block_expert, x, w, sw):
    grid_spec = pltpu.PrefetchScalarGridSpec(
        num_scalar_prefetch=2,
        grid=(NB,),
        in_specs=[
            pl.BlockSpec(memory_space=pl.ANY),
            pl.BlockSpec(memory_space=pl.ANY),
            pl.BlockSpec((1, 1, BM), lambda m, g, be: (m, 0, 0)),
        ],
        out_specs=pl.BlockSpec((BM, HIDDEN), lambda m, g, be: (m, 0)),
        scratch_shapes=[
            pltpu.VMEM((2, BM, INTER), jnp.float32),
            pltpu.VMEM((INTER, HIDDEN), jnp.float32),
            pltpu.VMEM((INTER, HIDDEN), jnp.bfloat16),
            pltpu.SemaphoreType.DMA((2,)),
            pltpu.SemaphoreType.DMA,
        ],
    )
    return pl.pallas_call(
        _mm_body,
        grid_spec=grid_spec,
        out_shape=jax.ShapeDtypeStruct((P, HIDDEN), jnp.float32),
        compiler_params=pltpu.CompilerParams(
            dimension_semantics=("arbitrary",),
            vmem_limit_bytes=100 * 1024 * 1024,
        ),
    )(gidx, block_expert, x, w, sw)


@functools.partial(
    pl.kernel,
    mesh=_mesh,
    out_type=jax.ShapeDtypeStruct((NTOKENS, HIDDEN), jnp.float32),
    scratch_types=[
        pltpu.VMEM((2 * TPW,), jnp.int32),
        pltpu.VMEM((2 * TCH, HIDDEN), jnp.float32),
        pltpu.VMEM((TCH, HIDDEN), jnp.float32),
        pltpu.SemaphoreType.DMA,
    ],
    compiler_params=_sc_params,
)
def _combine(y_hbm, pidx_hbm, out_hbm, idx_v, rows_v, out_v, sem):
    wid = lax.axis_index("s") * NC + lax.axis_index("c")
    tbase = wid * TPW
    pltpu.sync_copy(pidx_hbm.at[pl.ds(tbase * 2, 2 * TPW)], idx_v)

    def chunk(c, _):
        t0 = tbase + c * TCH
        pltpu.async_copy(
            y_hbm.at[idx_v.at[pl.ds(c * 2 * TCH, 2 * TCH)]], rows_v, sem)
        pltpu.make_async_copy(
            y_hbm.at[idx_v.at[pl.ds(c * 2 * TCH, 2 * TCH)]], rows_v, sem).wait()

        def jloop(j, _):
            off = j * 16
            for i in range(TCH):
                out_v[i, pl.ds(off, 16)] = (
                    rows_v[i, pl.ds(off, 16)] + rows_v[i + TCH, pl.ds(off, 16)]
                )
            return ()

        lax.fori_loop(0, HIDDEN // 16, jloop, ())
        pltpu.sync_copy(out_v, out_hbm.at[pl.ds(t0, TCH)])
        return ()

    lax.fori_loop(0, TPW // TCH, chunk, ())


def kernel(intermediate_states, w, topk_ids, topk_weight):
    flat_ids = topk_ids.reshape(-1)
    flat_w = topk_weight.reshape(-1)

    onehot = (flat_ids[:, None] == jnp.arange(NUM_EXPERTS, dtype=jnp.int32)[None, :])
    counts = jnp.sum(onehot.astype(jnp.int32), axis=0)
    padded = ((counts + BM - 1) // BM) * BM
    ends = jnp.cumsum(padded)
    starts = ends - padded
    cstart = jnp.concatenate(
        [jnp.zeros((1,), jnp.int32), jnp.cumsum(counts)[:-1].astype(jnp.int32)])
    shift = starts - cstart

    block_expert = jnp.clip(
        jnp.searchsorted(ends, jnp.arange(NB, dtype=jnp.int32) * BM, side="right"),
        0, NUM_EXPERTS - 1).astype(jnp.int32)

    order = jnp.argsort(flat_ids, stable=True).astype(jnp.int32)
    slot_e = jnp.repeat(block_expert, BM)
    s = jnp.arange(P, dtype=jnp.int32)
    srank = s - shift[slot_e]
    valid = s < starts[slot_e] + counts[slot_e]
    gidx = jnp.where(valid, order[jnp.clip(srank, 0, NROWS - 1)], -1)
    sw = jnp.where(valid, flat_w[jnp.maximum(gidx, 0)], 0.0)

    inv_order = jnp.argsort(order, stable=True).astype(jnp.int32)
    pos = inv_order + shift[flat_ids]

    pidx = pos.reshape(NTOKENS // TCH, TCH, TOPK).transpose(0, 2, 1).reshape(-1)
    pidx = pidx.astype(jnp.int32)

    y = _grouped_matmul(gidx, block_expert, intermediate_states, w,
                        sw.reshape(NB, 1, BM))

    return _combine(y, pidx)
